# baseline (speedup 1.0000x reference)
import jax
import jax.numpy as jnp
import numpy as np
from jax.experimental import pallas as pl

_B, _C, _H, _W = 16, 1024, 32, 32
_HW = _H * _W
_T = 1024

_KS0 = np.uint32(0)
_KS1 = np.uint32(42)
_KS2 = np.uint32(_KS0 ^ _KS1 ^ np.uint32(0x1BD11BDA))
_ROT = ((13, 15, 26, 6), (17, 29, 16, 24))


def _rounds(x0, x1, rs):
    for r in rs:
        x0 = x0 + x1
        x1 = (x1 << jnp.uint32(r)) | (x1 >> jnp.uint32(32 - r))
        x1 = x0 ^ x1
    return x0, x1


def _gumbel_block(base):
    c = jax.lax.broadcasted_iota(jnp.uint32, (_C, _T), 0)
    t = jax.lax.broadcasted_iota(jnp.uint32, (_C, _T), 1)
    cnt = base + c * jnp.uint32(_HW) + t

    x0 = jnp.full((_C, _T), _KS0, jnp.uint32)
    x1 = cnt + jnp.uint32(_KS1)
    x0, x1 = _rounds(x0, x1, _ROT[0])
    x0 = x0 + jnp.uint32(_KS1)
    x1 = x1 + jnp.uint32(_KS2 + np.uint32(1))
    x0, x1 = _rounds(x0, x1, _ROT[1])
    x0 = x0 + jnp.uint32(_KS2)
    x1 = x1 + jnp.uint32(_KS0 + np.uint32(2))
    x0, x1 = _rounds(x0, x1, _ROT[0])
    x0 = x0 + jnp.uint32(_KS0)
    x1 = x1 + jnp.uint32(_KS1 + np.uint32(3))
    x0, x1 = _rounds(x0, x1, _ROT[1])
    x0 = x0 + jnp.uint32(_KS1)
    x1 = x1 + jnp.uint32(_KS2 + np.uint32(4))
    x0, x1 = _rounds(x0, x1, _ROT[0])
    x0 = x0 + jnp.uint32(_KS2)
    x1 = x1 + jnp.uint32(_KS0 + np.uint32(5))

    bits = x0 ^ x1
    fb = (bits >> jnp.uint32(9)) | jnp.uint32(0x3F800000)
    f = jax.lax.bitcast_convert_type(fb, jnp.float32) - jnp.float32(1.0)
    tiny = jnp.float32(np.finfo(np.float32).tiny)
    span = jnp.float32(np.float32(1.0) - np.finfo(np.float32).tiny)
    u = jnp.maximum(tiny, f * span + tiny)
    return -jnp.log(-jnp.log(u))


def _onehot_argmax_kernel(x_ref, o_ref):
    b = pl.program_id(0).astype(jnp.uint32)
    j = pl.program_id(1).astype(jnp.uint32)
    base = b * jnp.uint32(_C * _HW) + j * jnp.uint32(_T)
    g = _gumbel_block(base)
    s = x_ref[0] + g
    idx = jnp.argmax(s, axis=0)
    iota = jax.lax.broadcasted_iota(jnp.int32, (_C, _T), 0)
    o_ref[0] = (iota == idx[None, :]).astype(jnp.float32)


def kernel(x):
    xr = x.reshape(_B, _C, _HW)
    out = pl.pallas_call(
        _onehot_argmax_kernel,
        grid=(_B, _HW // _T),
        in_specs=[
            pl.BlockSpec((1, _C, _T), lambda b, j: (b, 0, j)),
        ],
        out_specs=pl.BlockSpec((1, _C, _T), lambda b, j: (b, 0, j)),
        out_shape=jax.ShapeDtypeStruct((_B, _C, _HW), jnp.float32),
    )(xr)
    return out.reshape(_B, _C, _H, _W)


# speedup vs baseline: 1.2423x; 1.2423x over previous
import jax
import jax.numpy as jnp
import numpy as np
from jax.experimental import pallas as pl

_B, _C, _H, _W = 16, 1024, 32, 32
_HW = _H * _W
_T = 1024

_KS0 = np.uint32(0)
_KS1 = np.uint32(42)
_KS2 = np.uint32(_KS0 ^ _KS1 ^ np.uint32(0x1BD11BDA))
_ROT = ((13, 15, 26, 6), (17, 29, 16, 24))


def _rounds(x0, x1, rs):
    for r in rs:
        x0 = x0 + x1
        x1 = (x1 << jnp.uint32(r)) | (x1 >> jnp.uint32(32 - r))
        x1 = x0 ^ x1
    return x0, x1


def _gumbel_block(base):
    c = jax.lax.broadcasted_iota(jnp.uint32, (_C, _T), 0)
    t = jax.lax.broadcasted_iota(jnp.uint32, (_C, _T), 1)
    cnt = base + c * jnp.uint32(_HW) + t

    x0 = jnp.full((_C, _T), _KS0, jnp.uint32)
    x1 = cnt + jnp.uint32(_KS1)
    x0, x1 = _rounds(x0, x1, _ROT[0])
    x0 = x0 + jnp.uint32(_KS1)
    x1 = x1 + jnp.uint32(_KS2 + np.uint32(1))
    x0, x1 = _rounds(x0, x1, _ROT[1])
    x0 = x0 + jnp.uint32(_KS2)
    x1 = x1 + jnp.uint32(_KS0 + np.uint32(2))
    x0, x1 = _rounds(x0, x1, _ROT[0])
    x0 = x0 + jnp.uint32(_KS0)
    x1 = x1 + jnp.uint32(_KS1 + np.uint32(3))
    x0, x1 = _rounds(x0, x1, _ROT[1])
    x0 = x0 + jnp.uint32(_KS1)
    x1 = x1 + jnp.uint32(_KS2 + np.uint32(4))
    x0, x1 = _rounds(x0, x1, _ROT[0])
    x0 = x0 + jnp.uint32(_KS2)
    x1 = x1 + jnp.uint32(_KS0 + np.uint32(5))

    bits = x0 ^ x1
    fb = (bits >> jnp.uint32(9)) | jnp.uint32(0x3F800000)
    f = jax.lax.bitcast_convert_type(fb, jnp.float32) - jnp.float32(1.0)
    tiny = jnp.float32(np.finfo(np.float32).tiny)
    span = jnp.float32(np.float32(1.0) - np.finfo(np.float32).tiny)
    u = jnp.maximum(tiny, f * span + tiny)
    return -jnp.log(-jnp.log(u))


def _onehot_argmax_kernel(x_ref, g_ref, o_ref):
    s = x_ref[0] + g_ref[0]
    idx = jnp.argmax(s, axis=0)
    iota = jax.lax.broadcasted_iota(jnp.int32, (_C, _T), 0)
    o_ref[0] = (iota == idx[None, :]).astype(jnp.float32)


def kernel(x):
    g = jax.random.gumbel(jax.random.key(42), (_B, _C, _H, _W),
                          dtype=jnp.float32).reshape(_B, _C, _HW)
    xr = x.reshape(_B, _C, _HW)
    out = pl.pallas_call(
        _onehot_argmax_kernel,
        grid=(_B, _HW // _T),
        in_specs=[
            pl.BlockSpec((1, _C, _T), lambda b, j: (b, 0, j)),
            pl.BlockSpec((1, _C, _T), lambda b, j: (b, 0, j)),
        ],
        out_specs=pl.BlockSpec((1, _C, _T), lambda b, j: (b, 0, j)),
        out_shape=jax.ShapeDtypeStruct((_B, _C, _HW), jnp.float32),
    )(xr, g)
    return out.reshape(_B, _C, _H, _W)
